# baseline (speedup 1.0000x reference)
import math

import jax
import jax.numpy as jnp
from jax.experimental import pallas as pl
from jax.experimental.pallas import tpu as pltpu

_SQRT2 = math.sqrt(2.0)


def _gelu(x):
    return 0.5 * x * (1.0 + jax.lax.erf(x / _SQRT2))


def _dot(a, b):
    return jax.lax.dot(a, b, preferred_element_type=jnp.float32)


def _ln(x, g, b, eps):
    m = jnp.mean(x, axis=-1, keepdims=True)
    v = jnp.mean((x - m) ** 2, axis=-1, keepdims=True)
    return (x - m) * jax.lax.rsqrt(v + eps) * g + b


def _full_spec(shape):
    return pl.BlockSpec(shape, lambda i: (0,) * len(shape))


def _bspec(shape, bdim, tb):
    bs = tuple(tb if d == bdim else shape[d] for d in range(len(shape)))
    return pl.BlockSpec(bs, lambda i: tuple(i if d == bdim else 0
                                            for d in range(len(shape))))


_BLK_KEYS = ('dw', 'dwb', 'lg', 'lb', 'w1', 'b1', 'w2', 'b2', 'g')


def _blk_args(bp):
    C = bp['dwb'].shape[-1]
    return (bp['dw'].reshape(49, C), bp['dwb'].reshape(1, C),
            bp['lg'].reshape(1, C), bp['lb'].reshape(1, C), bp['w1'],
            bp['b1'].reshape(1, -1), bp['w2'], bp['b2'].reshape(1, C),
            bp['g'].reshape(1, C))


def _blk_tail(x, y, dwb, lg, lb, w1, b1, w2, b2, g):
    H, W, tb, C = x.shape
    y = y + dwb
    y = _ln(y, lg, lb, 1e-6)
    h = _gelu(_dot(y.reshape(H * W * tb, C), w1) + b1)
    o = _dot(h, w2) + b2
    return x + g * o.reshape(H, W, tb, C)


def _blk_taps(x, pad_ref, dwk, dwb, lg, lb, w1, b1, w2, b2, g):
    H, W, tb, C = x.shape
    pad_ref[...] = jnp.zeros(pad_ref.shape, jnp.float32)
    pad_ref[3:H + 3, 3:W + 3, :, :] = x
    rows = []
    for h in range(H):
        acc = jnp.zeros((W, tb, C), jnp.float32)
        for dh in range(7):
            row = pad_ref[h + dh]
            for dw in range(7):
                acc = acc + row[dw:dw + W] * dwk[dh * 7 + dw]
        rows.append(acc)
    y = jnp.stack(rows)
    return _blk_tail(x, y, dwb, lg, lb, w1, b1, w2, b2, g)


def _blk_pairs(x, kq, dwb, lg, lb, w1, b1, w2, b2, g):
    H, W, tb, C = x.shape
    y = jnp.zeros((H, W, tb, C), jnp.float32)
    for q in range(H * W):
        qh, qw = divmod(q, W)
        y = y + x[qh, qw][None, None] * kq[q]
    return _blk_tail(x, y, dwb, lg, lb, w1, b1, w2, b2, g)


def _kq(bp, H, W):
    C = bp['dwb'].shape[-1]
    k2 = bp['dw'].reshape(7, 7, C)
    rows = []
    for q in range(H * W):
        qh, qw = divmod(q, W)
        row = jnp.stack([jnp.stack([k2[qh - ph + 3, qw - pw + 3]
                                    for pw in range(W)]) for ph in range(H)])
        rows.append(row[:, :, None, :])
    return jnp.stack(rows)


def _down_v(x, lg, lb, w4, b):
    H, W, tb, C = x.shape
    Cout = w4.shape[-1]
    xl = _ln(x, lg, lb, 1e-6)
    xp = xl.reshape(H // 2, 2, W // 2, 2, tb, C)
    acc = jnp.zeros((H // 2 * (W // 2) * tb, Cout), jnp.float32)
    for j in range(4):
        dh, dw = divmod(j, 2)
        acc = acc + _dot(xp[:, dh, :, dw].reshape(-1, C), w4[j])
    return (acc + b).reshape(H // 2, W // 2, tb, Cout)


def _down_args(d):
    C, Cout = d['w'].shape[2], d['w'].shape[3]
    return (d['lg'].reshape(1, C), d['lb'].reshape(1, C),
            d['w'].reshape(4, C, Cout), d['b'].reshape(1, Cout))


def _stage1_call(xp, p, B):
    TB = 16
    blocks = p['stages'][0]
    args = [xp, p['stem_w'].reshape(64, 96), p['stem_b'].reshape(1, 96),
            p['stem_lg'].reshape(1, 96), p['stem_lb'].reshape(1, 96)]
    for bp in blocks:
        args.extend(_blk_args(bp))
    args.extend(_down_args(p['downs'][0]))

    def body(*refs):
        xp_ref, sw, sb, slg, slb = refs[:5]
        o_ref, pad_ref = refs[-2], refs[-1]
        P, tb, K = xp_ref.shape
        y = _dot(xp_ref[...].reshape(P * tb, K), sw[...]) + sb[...]
        y = _ln(y, slg[...], slb[...], 1e-6)
        x = y.reshape(16, 16, tb, 96)
        for bi in range(3):
            br = refs[5 + 9 * bi:5 + 9 * (bi + 1)]
            x = _blk_taps(x, pad_ref, *[r[...] for r in br])
        dr = refs[5 + 27:5 + 31]
        o_ref[...] = _down_v(x, *[r[...] for r in dr])

    return pl.pallas_call(
        body,
        grid=(B // TB,),
        in_specs=[_bspec(xp.shape, 1, TB)] + [_full_spec(a.shape)
                                              for a in args[1:]],
        out_specs=_bspec((8, 8, B, 192), 2, TB),
        out_shape=jax.ShapeDtypeStruct((8, 8, B, 192), jnp.float32),
        scratch_shapes=[pltpu.VMEM((22, 22, TB, 96), jnp.float32)],
    )(*args)


def _stage2_call(x, p, B):
    TB = 32
    blocks = p['stages'][1]
    args = [x]
    for bp in blocks:
        args.extend(_blk_args(bp))
    args.extend(_down_args(p['downs'][1]))

    def body(*refs):
        x_ref = refs[0]
        o_ref, pad_ref = refs[-2], refs[-1]
        x = x_ref[...]
        for bi in range(3):
            br = refs[1 + 9 * bi:1 + 9 * (bi + 1)]
            x = _blk_taps(x, pad_ref, *[r[...] for r in br])
        dr = refs[1 + 27:1 + 31]
        o_ref[...] = _down_v(x, *[r[...] for r in dr])

    return pl.pallas_call(
        body,
        grid=(B // TB,),
        in_specs=[_bspec(x.shape, 2, TB)] + [_full_spec(a.shape)
                                             for a in args[1:]],
        out_specs=_bspec((4, 4, B, 384), 2, TB),
        out_shape=jax.ShapeDtypeStruct((4, 4, B, 384), jnp.float32),
        scratch_shapes=[pltpu.VMEM((14, 14, TB, 192), jnp.float32)],
    )(*args)


def _stage3_call(x, p, B):
    TB = 64
    blocks = p['stages'][2]

    def run(x, blks, down):
        C = 384
        args = [x]
        for bp in blks:
            a = list(_blk_args(bp))
            a[0] = _kq(bp, 4, 4)
            args.extend(a)
        if down is not None:
            args.extend(_down_args(down))
            oshape = (2, 2, B, 768)
        else:
            oshape = (4, 4, B, C)
        nb = len(blks)

        def body(*refs):
            x_ref, o_ref = refs[0], refs[-1]
            x = x_ref[...]
            for bi in range(nb):
                br = refs[1 + 9 * bi:1 + 9 * (bi + 1)]
                x = _blk_pairs(x, *[r[...] for r in br])
            if down is not None:
                dr = refs[1 + 9 * nb:1 + 9 * nb + 4]
                x = _down_v(x, *[r[...] for r in dr])
            o_ref[...] = x

        return pl.pallas_call(
            body,
            grid=(B // TB,),
            in_specs=[_bspec(x.shape, 2, TB)] + [_full_spec(a.shape)
                                                 for a in args[1:]],
            out_specs=_bspec(oshape, 2, TB),
            out_shape=jax.ShapeDtypeStruct(oshape, jnp.float32),
        )(*args)

    x = run(x, blocks[:5], None)
    return run(x, blocks[5:], p['downs'][2])


def _stage4_call(x, p, B):
    TB = 64

    def body(*refs):
        x_ref, o_ref = refs[0], refs[-1]
        o_ref[...] = _blk_pairs(x_ref[...], *[r[...] for r in refs[1:-1]])

    for bp in p['stages'][3]:
        a = list(_blk_args(bp))
        a[0] = _kq(bp, 2, 2)
        args = [x] + a
        x = pl.pallas_call(
            body,
            grid=(B // TB,),
            in_specs=[_bspec(x.shape, 2, TB)] + [_full_spec(w.shape)
                                                 for w in args[1:]],
            out_specs=_bspec((2, 2, B, 768), 2, TB),
            out_shape=jax.ShapeDtypeStruct((2, 2, B, 768), jnp.float32),
        )(*args)
    return x


def _head_body(x_ref, hlg_ref, hlb_ref, hmlg_ref, hmlb_ref, w1_ref, b1_ref,
               w2_ref, b2_ref, w3_ref, b3_ref, halg_ref, halb_ref, wa_ref,
               ba_ref, o_ref):
    f = jnp.mean(x_ref[...], axis=(0, 1))
    f = _ln(f, hlg_ref[...], hlb_ref[...], 1e-6)
    h = _gelu(f)
    h = _ln(h, hmlg_ref[...], hmlb_ref[...], 1e-5)
    h = jax.nn.relu(_dot(h, w1_ref[...]) + b1_ref[...])
    h = _dot(h, w2_ref[...]) + b2_ref[...]
    h = _dot(h, w3_ref[...]) + b3_ref[...]
    a = jnp.tanh(_dot(_ln(f, halg_ref[...], halb_ref[...], 1e-5),
                      wa_ref[...]) + ba_ref[...])
    o_ref[...] = h * a


def _head_call(x, p, bbp):
    B = x.shape[2]
    r = lambda a: a.reshape(1, -1)
    args = (x, r(bbp['head_lg']), r(bbp['head_lb']), r(p['hm_lg']),
            r(p['hm_lb']), p['hm_w1'], r(p['hm_b1']), p['hm_w2'],
            r(p['hm_b2']), p['hm_w3'], r(p['hm_b3']), r(p['ha_lg']),
            r(p['ha_lb']), p['ha_w'], r(p['ha_b']))
    return pl.pallas_call(
        _head_body,
        grid=(1,),
        in_specs=[_full_spec(a.shape) for a in args],
        out_specs=_full_spec((B, 128)),
        out_shape=jax.ShapeDtypeStruct((B, 128), jnp.float32),
    )(*args)


_TOWER_KEYS = ('ws', 'bs', 'lag', 'lab', 'wa', 'ba', 'lmg', 'lmb', 'wm',
               'bm', 'wk', 'bk')


def _tower_args(p):
    r = lambda a: a.reshape(1, -1)
    return (p['ws'], r(p['bs']), r(p['lag']), r(p['lab']), p['wa'],
            r(p['ba']), r(p['lmg']), r(p['lmb']), p['wm'], r(p['bm']),
            p['wk'], r(p['bk']))


def _tower_v(x, ws, bs, lag, lab, wa, ba, lmg, lmb, wm, bm, wk, bk):
    h = _gelu(_dot(x, ws) + bs)
    gate = jax.nn.sigmoid(_dot(_ln(h, lag, lab, 1e-5), wa) + ba)
    m = _dot(_ln(h, lmg, lmb, 1e-5), wm) + bm
    return m * gate + (_dot(x, wk) + bk)


def _tower_body(x_ref, *refs):
    o_ref = refs[-1]
    o_ref[...] = _tower_v(x_ref[...], *[r[...] for r in refs[:-1]])


def _tower_call(x, p):
    B = x.shape[0]
    O = p['wa'].shape[1]
    args = (x,) + _tower_args(p)
    return pl.pallas_call(
        _tower_body,
        grid=(1,),
        in_specs=[_full_spec(a.shape) for a in args],
        out_specs=_full_spec((B, O)),
        out_shape=jax.ShapeDtypeStruct((B, O), jnp.float32),
    )(*args)


def _moe_body(*refs):
    f_ref, rw1_ref, rb1_ref, rw2_ref, rb2_ref = refs[:5]
    o_ref = refs[-1]
    feats = f_ref[...]
    B = feats.shape[0]
    r = jnp.tanh(_dot(feats, rw1_ref[...]) + rb1_ref[...])
    fw = jax.nn.sigmoid(_dot(r, rw2_ref[...]) + rb2_ref[...])
    col = jax.lax.broadcasted_iota(jnp.int32, fw.shape, 1)
    w0 = jnp.max(fw, axis=1, keepdims=True)
    i0 = jnp.min(jnp.where(fw == w0, col, 127), axis=1, keepdims=True)
    first0 = col == i0
    fw2 = jnp.where(first0, -1.0, fw)
    w1 = jnp.max(fw2, axis=1, keepdims=True)
    i1 = jnp.min(jnp.where(fw2 == w1, col, 127), axis=1, keepdims=True)
    first1 = col == i1
    gates = jnp.where(first0, w0, 0.0) + jnp.where(first1, w1, 0.0)
    moe = jnp.zeros((B, o_ref.shape[1]), jnp.float32)
    for e in range(8):
        er = refs[5 + 12 * e:5 + 12 * (e + 1)]
        eo = _tower_v(feats, *[r_[...] for r_ in er])
        moe = moe + gates[:, e:e + 1] * eo
    o_ref[...] = moe


def _moe_call(feats, params):
    B = feats.shape[0]
    args = [feats, params['r_w1'], params['r_b1'].reshape(1, -1),
            params['r_w2'], params['r_b2'].reshape(1, -1)]
    for e in params['experts']:
        args.extend(_tower_args(e))
    return pl.pallas_call(
        _moe_body,
        grid=(1,),
        in_specs=[_full_spec(a.shape) for a in args],
        out_specs=_full_spec((B, 5)),
        out_shape=jax.ShapeDtypeStruct((B, 5), jnp.float32),
    )(*args)


def kernel(metadata, image, params):
    md = metadata
    idx = lambda cols: md[:, jnp.array(cols)]
    B = image.shape[0]
    bb = params['img']['bb']
    xp = image.reshape(B, 4, 16, 4, 16, 4)
    xp = jnp.transpose(xp, (2, 4, 0, 3, 5, 1)).reshape(256, B, 64)
    x = _stage1_call(xp, bb, B)
    x = _stage2_call(x, bb, B)
    x = _stage3_call(x, bb, B)
    x4 = _stage4_call(x, bb, B)
    img_f = _head_call(x4, params['img'], bb)
    nsta = _tower_call(idx([0, 2]), params['nst1'])
    nstb = _tower_call(idx([1, 3]), params['nst2'])
    spatial = _tower_call(idx([2, 3, 4]), params['spatial'])
    psf = _tower_call(idx([5, 14]), params['psf'])
    mag = _tower_call(idx([6, 9, 10, 13, 15, 17, 18]), params['mag'])
    coord = _tower_call(idx([7, 8]), params['coord'])
    mega = _tower_call(md[:, :19], params['mega'])
    lc = _tower_call(idx([6, 9, 10, 13, 15, 17, 18, 19, 20, 21, 22, 23]),
                     params['lc'])
    feats = jnp.concatenate([nsta, nstb, spatial, psf, mag, coord, mega,
                             img_f, lc], axis=1)
    return _moe_call(feats, params)


# speedup vs baseline: 1.0102x; 1.0102x over previous
import math

import jax
import jax.numpy as jnp
from jax.experimental import pallas as pl
from jax.experimental.pallas import tpu as pltpu

_SQRT2 = math.sqrt(2.0)


def _gelu(x):
    return 0.5 * x * (1.0 + jax.lax.erf(x / _SQRT2))


def _dot(a, b):
    return jax.lax.dot(a, b, preferred_element_type=jnp.float32)


def _ln(x, g, b, eps):
    m = jnp.mean(x, axis=-1, keepdims=True)
    v = jnp.mean((x - m) ** 2, axis=-1, keepdims=True)
    return (x - m) * jax.lax.rsqrt(v + eps) * g + b


def _full_spec(shape):
    return pl.BlockSpec(shape, lambda i: (0,) * len(shape))


def _bspec(shape, bdim, tb):
    bs = tuple(tb if d == bdim else shape[d] for d in range(len(shape)))
    return pl.BlockSpec(bs, lambda i: tuple(i if d == bdim else 0
                                            for d in range(len(shape))))


_BLK_KEYS = ('dw', 'dwb', 'lg', 'lb', 'w1', 'b1', 'w2', 'b2', 'g')


def _blk_args(bp):
    C = bp['dwb'].shape[-1]
    return (bp['dw'].reshape(49, C), bp['dwb'].reshape(1, C),
            bp['lg'].reshape(1, C), bp['lb'].reshape(1, C), bp['w1'],
            bp['b1'].reshape(1, -1), bp['w2'], bp['b2'].reshape(1, C),
            bp['g'].reshape(1, C))


def _blk_tail(x, y, dwb, lg, lb, w1, b1, w2, b2, g):
    H, W, tb, C = x.shape
    y = y + dwb
    y = _ln(y, lg, lb, 1e-6)
    h = _gelu(_dot(y.reshape(H * W * tb, C), w1) + b1)
    o = _dot(h, w2) + b2
    return x + g * o.reshape(H, W, tb, C)


def _blk_taps(x, pad_ref, dwk, dwb, lg, lb, w1, b1, w2, b2, g):
    H, W, tb, C = x.shape
    pad_ref[...] = jnp.zeros(pad_ref.shape, jnp.float32)
    pad_ref[3:H + 3, 3:W + 3, :, :] = x
    y = jnp.zeros((H, W, tb, C), jnp.float32)
    for k in range(49):
        dh, dw = divmod(k, 7)
        y = y + pad_ref[dh:dh + H, dw:dw + W, :, :] * dwk[k]
    return _blk_tail(x, y, dwb, lg, lb, w1, b1, w2, b2, g)


def _blk_pairs(x, kq, dwb, lg, lb, w1, b1, w2, b2, g):
    H, W, tb, C = x.shape
    y = jnp.zeros((H, W, tb, C), jnp.float32)
    for q in range(H * W):
        qh, qw = divmod(q, W)
        y = y + x[qh, qw][None, None] * kq[q]
    return _blk_tail(x, y, dwb, lg, lb, w1, b1, w2, b2, g)


def _kq(bp, H, W):
    C = bp['dwb'].shape[-1]
    k2 = bp['dw'].reshape(7, 7, C)
    rows = []
    for q in range(H * W):
        qh, qw = divmod(q, W)
        row = jnp.stack([jnp.stack([k2[qh - ph + 3, qw - pw + 3]
                                    for pw in range(W)]) for ph in range(H)])
        rows.append(row[:, :, None, :])
    return jnp.stack(rows)


def _down_v(x, lg, lb, w4, b):
    H, W, tb, C = x.shape
    Cout = w4.shape[-1]
    xl = _ln(x, lg, lb, 1e-6)
    xp = xl.reshape(H // 2, 2, W // 2, 2, tb, C)
    acc = jnp.zeros((H // 2 * (W // 2) * tb, Cout), jnp.float32)
    for j in range(4):
        dh, dw = divmod(j, 2)
        acc = acc + _dot(xp[:, dh, :, dw].reshape(-1, C), w4[j])
    return (acc + b).reshape(H // 2, W // 2, tb, Cout)


def _down_args(d):
    C, Cout = d['w'].shape[2], d['w'].shape[3]
    return (d['lg'].reshape(1, C), d['lb'].reshape(1, C),
            d['w'].reshape(4, C, Cout), d['b'].reshape(1, Cout))


def _stage1_call(xp, p, B):
    TB = 16
    blocks = p['stages'][0]
    args = [xp, p['stem_w'].reshape(64, 96), p['stem_b'].reshape(1, 96),
            p['stem_lg'].reshape(1, 96), p['stem_lb'].reshape(1, 96)]
    for bp in blocks:
        args.extend(_blk_args(bp))
    args.extend(_down_args(p['downs'][0]))

    def body(*refs):
        xp_ref, sw, sb, slg, slb = refs[:5]
        o_ref, pad_ref = refs[-2], refs[-1]
        P, tb, K = xp_ref.shape
        y = _dot(xp_ref[...].reshape(P * tb, K), sw[...]) + sb[...]
        y = _ln(y, slg[...], slb[...], 1e-6)
        x = y.reshape(16, 16, tb, 96)
        for bi in range(3):
            br = refs[5 + 9 * bi:5 + 9 * (bi + 1)]
            x = _blk_taps(x, pad_ref, *[r[...] for r in br])
        dr = refs[5 + 27:5 + 31]
        o_ref[...] = _down_v(x, *[r[...] for r in dr])

    return pl.pallas_call(
        body,
        grid=(B // TB,),
        in_specs=[_bspec(xp.shape, 1, TB)] + [_full_spec(a.shape)
                                              for a in args[1:]],
        out_specs=_bspec((8, 8, B, 192), 2, TB),
        out_shape=jax.ShapeDtypeStruct((8, 8, B, 192), jnp.float32),
        scratch_shapes=[pltpu.VMEM((22, 22, TB, 96), jnp.float32)],
    )(*args)


def _stage2_call(x, p, B):
    TB = 32
    blocks = p['stages'][1]
    args = [x]
    for bp in blocks:
        args.extend(_blk_args(bp))
    args.extend(_down_args(p['downs'][1]))

    def body(*refs):
        x_ref = refs[0]
        o_ref, pad_ref = refs[-2], refs[-1]
        x = x_ref[...]
        for bi in range(3):
            br = refs[1 + 9 * bi:1 + 9 * (bi + 1)]
            x = _blk_taps(x, pad_ref, *[r[...] for r in br])
        dr = refs[1 + 27:1 + 31]
        o_ref[...] = _down_v(x, *[r[...] for r in dr])

    return pl.pallas_call(
        body,
        grid=(B // TB,),
        in_specs=[_bspec(x.shape, 2, TB)] + [_full_spec(a.shape)
                                             for a in args[1:]],
        out_specs=_bspec((4, 4, B, 384), 2, TB),
        out_shape=jax.ShapeDtypeStruct((4, 4, B, 384), jnp.float32),
        scratch_shapes=[pltpu.VMEM((14, 14, TB, 192), jnp.float32)],
    )(*args)


def _stage3_call(x, p, B):
    TB = 64
    blocks = p['stages'][2]

    def run(x, blks, down):
        C = 384
        args = [x]
        for bp in blks:
            a = list(_blk_args(bp))
            a[0] = _kq(bp, 4, 4)
            args.extend(a)
        if down is not None:
            args.extend(_down_args(down))
            oshape = (2, 2, B, 768)
        else:
            oshape = (4, 4, B, C)
        nb = len(blks)

        def body(*refs):
            x_ref, o_ref = refs[0], refs[-1]
            x = x_ref[...]
            for bi in range(nb):
                br = refs[1 + 9 * bi:1 + 9 * (bi + 1)]
                x = _blk_pairs(x, *[r[...] for r in br])
            if down is not None:
                dr = refs[1 + 9 * nb:1 + 9 * nb + 4]
                x = _down_v(x, *[r[...] for r in dr])
            o_ref[...] = x

        return pl.pallas_call(
            body,
            grid=(B // TB,),
            in_specs=[_bspec(x.shape, 2, TB)] + [_full_spec(a.shape)
                                                 for a in args[1:]],
            out_specs=_bspec(oshape, 2, TB),
            out_shape=jax.ShapeDtypeStruct(oshape, jnp.float32),
        )(*args)

    x = run(x, blocks[:5], None)
    return run(x, blocks[5:], p['downs'][2])


def _stage4_call(x, p, B):
    TB = 64

    def body(*refs):
        x_ref, o_ref = refs[0], refs[-1]
        o_ref[...] = _blk_pairs(x_ref[...], *[r[...] for r in refs[1:-1]])

    for bp in p['stages'][3]:
        a = list(_blk_args(bp))
        a[0] = _kq(bp, 2, 2)
        args = [x] + a
        x = pl.pallas_call(
            body,
            grid=(B // TB,),
            in_specs=[_bspec(x.shape, 2, TB)] + [_full_spec(w.shape)
                                                 for w in args[1:]],
            out_specs=_bspec((2, 2, B, 768), 2, TB),
            out_shape=jax.ShapeDtypeStruct((2, 2, B, 768), jnp.float32),
        )(*args)
    return x


def _head_body(x_ref, hlg_ref, hlb_ref, hmlg_ref, hmlb_ref, w1_ref, b1_ref,
               w2_ref, b2_ref, w3_ref, b3_ref, halg_ref, halb_ref, wa_ref,
               ba_ref, o_ref):
    f = jnp.mean(x_ref[...], axis=(0, 1))
    f = _ln(f, hlg_ref[...], hlb_ref[...], 1e-6)
    h = _gelu(f)
    h = _ln(h, hmlg_ref[...], hmlb_ref[...], 1e-5)
    h = jax.nn.relu(_dot(h, w1_ref[...]) + b1_ref[...])
    h = _dot(h, w2_ref[...]) + b2_ref[...]
    h = _dot(h, w3_ref[...]) + b3_ref[...]
    a = jnp.tanh(_dot(_ln(f, halg_ref[...], halb_ref[...], 1e-5),
                      wa_ref[...]) + ba_ref[...])
    o_ref[...] = h * a


def _head_call(x, p, bbp):
    B = x.shape[2]
    r = lambda a: a.reshape(1, -1)
    args = (x, r(bbp['head_lg']), r(bbp['head_lb']), r(p['hm_lg']),
            r(p['hm_lb']), p['hm_w1'], r(p['hm_b1']), p['hm_w2'],
            r(p['hm_b2']), p['hm_w3'], r(p['hm_b3']), r(p['ha_lg']),
            r(p['ha_lb']), p['ha_w'], r(p['ha_b']))
    return pl.pallas_call(
        _head_body,
        grid=(1,),
        in_specs=[_full_spec(a.shape) for a in args],
        out_specs=_full_spec((B, 128)),
        out_shape=jax.ShapeDtypeStruct((B, 128), jnp.float32),
    )(*args)


_TOWER_KEYS = ('ws', 'bs', 'lag', 'lab', 'wa', 'ba', 'lmg', 'lmb', 'wm',
               'bm', 'wk', 'bk')


def _tower_args(p):
    r = lambda a: a.reshape(1, -1)
    return (p['ws'], r(p['bs']), r(p['lag']), r(p['lab']), p['wa'],
            r(p['ba']), r(p['lmg']), r(p['lmb']), p['wm'], r(p['bm']),
            p['wk'], r(p['bk']))


def _tower_v(x, ws, bs, lag, lab, wa, ba, lmg, lmb, wm, bm, wk, bk):
    h = _gelu(_dot(x, ws) + bs)
    gate = jax.nn.sigmoid(_dot(_ln(h, lag, lab, 1e-5), wa) + ba)
    m = _dot(_ln(h, lmg, lmb, 1e-5), wm) + bm
    return m * gate + (_dot(x, wk) + bk)


def _tower_body(x_ref, *refs):
    o_ref = refs[-1]
    o_ref[...] = _tower_v(x_ref[...], *[r[...] for r in refs[:-1]])


def _tower_call(x, p):
    B = x.shape[0]
    O = p['wa'].shape[1]
    args = (x,) + _tower_args(p)
    return pl.pallas_call(
        _tower_body,
        grid=(1,),
        in_specs=[_full_spec(a.shape) for a in args],
        out_specs=_full_spec((B, O)),
        out_shape=jax.ShapeDtypeStruct((B, O), jnp.float32),
    )(*args)


def _moe_body(*refs):
    f_ref, rw1_ref, rb1_ref, rw2_ref, rb2_ref = refs[:5]
    o_ref = refs[-1]
    feats = f_ref[...]
    B = feats.shape[0]
    r = jnp.tanh(_dot(feats, rw1_ref[...]) + rb1_ref[...])
    fw = jax.nn.sigmoid(_dot(r, rw2_ref[...]) + rb2_ref[...])
    col = jax.lax.broadcasted_iota(jnp.int32, fw.shape, 1)
    w0 = jnp.max(fw, axis=1, keepdims=True)
    i0 = jnp.min(jnp.where(fw == w0, col, 127), axis=1, keepdims=True)
    first0 = col == i0
    fw2 = jnp.where(first0, -1.0, fw)
    w1 = jnp.max(fw2, axis=1, keepdims=True)
    i1 = jnp.min(jnp.where(fw2 == w1, col, 127), axis=1, keepdims=True)
    first1 = col == i1
    gates = jnp.where(first0, w0, 0.0) + jnp.where(first1, w1, 0.0)
    moe = jnp.zeros((B, o_ref.shape[1]), jnp.float32)
    for e in range(8):
        er = refs[5 + 12 * e:5 + 12 * (e + 1)]
        eo = _tower_v(feats, *[r_[...] for r_ in er])
        moe = moe + gates[:, e:e + 1] * eo
    o_ref[...] = moe


def _moe_call(feats, params):
    B = feats.shape[0]
    args = [feats, params['r_w1'], params['r_b1'].reshape(1, -1),
            params['r_w2'], params['r_b2'].reshape(1, -1)]
    for e in params['experts']:
        args.extend(_tower_args(e))
    return pl.pallas_call(
        _moe_body,
        grid=(1,),
        in_specs=[_full_spec(a.shape) for a in args],
        out_specs=_full_spec((B, 5)),
        out_shape=jax.ShapeDtypeStruct((B, 5), jnp.float32),
    )(*args)


def kernel(metadata, image, params):
    md = metadata
    idx = lambda cols: md[:, jnp.array(cols)]
    B = image.shape[0]
    bb = params['img']['bb']
    xp = image.reshape(B, 4, 16, 4, 16, 4)
    xp = jnp.transpose(xp, (2, 4, 0, 3, 5, 1)).reshape(256, B, 64)
    x = _stage1_call(xp, bb, B)
    x = _stage2_call(x, bb, B)
    x = _stage3_call(x, bb, B)
    x4 = _stage4_call(x, bb, B)
    img_f = _head_call(x4, params['img'], bb)
    nsta = _tower_call(idx([0, 2]), params['nst1'])
    nstb = _tower_call(idx([1, 3]), params['nst2'])
    spatial = _tower_call(idx([2, 3, 4]), params['spatial'])
    psf = _tower_call(idx([5, 14]), params['psf'])
    mag = _tower_call(idx([6, 9, 10, 13, 15, 17, 18]), params['mag'])
    coord = _tower_call(idx([7, 8]), params['coord'])
    mega = _tower_call(md[:, :19], params['mega'])
    lc = _tower_call(idx([6, 9, 10, 13, 15, 17, 18, 19, 20, 21, 22, 23]),
                     params['lc'])
    feats = jnp.concatenate([nsta, nstb, spatial, psf, mag, coord, mega,
                             img_f, lc], axis=1)
    return _moe_call(feats, params)
